# Initial kernel scaffold; baseline (speedup 1.0000x reference)
import functools

import jax
import jax.numpy as jnp
from jax import lax
from jax.experimental import pallas as pl
from jax.experimental.pallas import tpu as pltpu
from jax.experimental.pallas import tpu_sc as plsc

N = 50000
NP = 51200
E = 1600000
D_IN = 512
D = 256

NSC = 2
NSUB = 16
LANES = 16

WN = 6400
WPC = 4
ACC_ROWS = WN + 8
ECHUNK = 4000
GQ = 64

HCHUNK = 2000
HROWS_PER_SUB = NP // NSUB

_vmesh = plsc.VectorSubcoreMesh(core_axis_name="c", subcore_axis_name="s")


def _reducer_body(x_ref, w_ref, b_ref, o_ref):
    o_ref[...] = (
        jnp.dot(x_ref[...], w_ref[...], preferred_element_type=jnp.float32)
        + b_ref[...]
    )


def _tc_reducer(x, wr, br):
    return pl.pallas_call(
        _reducer_body,
        grid=(125,),
        in_specs=[
            pl.BlockSpec((400, D_IN), lambda i: (i, 0)),
            pl.BlockSpec((D_IN, D), lambda i: (0, 0)),
            pl.BlockSpec((1, D), lambda i: (0, 0)),
        ],
        out_specs=pl.BlockSpec((400, D), lambda i: (i, 0)),
        out_shape=jax.ShapeDtypeStruct((NP, D), jnp.float32),
    )(x, wr, br)


def _scale_mm_body(h_ref, ns_ref, w_ref, o_ref):
    o_ref[...] = jnp.dot(
        h_ref[...] * ns_ref[...], w_ref[...],
        preferred_element_type=jnp.float32)


def _tc_scale_matmul(h, ns, w):
    return pl.pallas_call(
        _scale_mm_body,
        grid=(100,),
        in_specs=[
            pl.BlockSpec((512, D), lambda i: (i, 0)),
            pl.BlockSpec((512, 1), lambda i: (i, 0)),
            pl.BlockSpec((D, D), lambda i: (0, 0)),
        ],
        out_specs=pl.BlockSpec((512, D), lambda i: (i, 0)),
        out_shape=jax.ShapeDtypeStruct((NP, D), jnp.float32),
    )(h, ns, w)


def _post_mm_body(a_ref, nd_ref, b_ref, ns_ref, w_ref, o_ref):
    z = jax.nn.relu(a_ref[...] * nd_ref[...] + b_ref[...])
    o_ref[...] = jnp.dot(z * ns_ref[...], w_ref[...],
                         preferred_element_type=jnp.float32)


def _tc_post_matmul(agg, nd, b, ns, w):
    return pl.pallas_call(
        _post_mm_body,
        grid=(100,),
        in_specs=[
            pl.BlockSpec((512, D), lambda i: (i, 0)),
            pl.BlockSpec((512, 1), lambda i: (i, 0)),
            pl.BlockSpec((1, D), lambda i: (0, 0)),
            pl.BlockSpec((512, 1), lambda i: (i, 0)),
            pl.BlockSpec((D, D), lambda i: (0, 0)),
        ],
        out_specs=pl.BlockSpec((512, D), lambda i: (i, 0)),
        out_shape=jax.ShapeDtypeStruct((NP, D), jnp.float32),
    )(agg, nd, b, ns, w)


def _uv_body(a_ref, nd_ref, b_ref, wt_ref, wb_ref, be_ref, u_ref, v_ref):
    z = jax.nn.relu(a_ref[...] * nd_ref[...] + b_ref[...])
    u_ref[...] = jnp.sum(z * wt_ref[...], axis=1, keepdims=True) + be_ref[0, 0]
    v_ref[...] = jnp.sum(z * wb_ref[...], axis=1, keepdims=True)


def _tc_uv(agg, nd, b, we_top, we_bot, be):
    return pl.pallas_call(
        _uv_body,
        grid=(100,),
        in_specs=[
            pl.BlockSpec((512, D), lambda i: (i, 0)),
            pl.BlockSpec((512, 1), lambda i: (i, 0)),
            pl.BlockSpec((1, D), lambda i: (0, 0)),
            pl.BlockSpec((1, D), lambda i: (0, 0)),
            pl.BlockSpec((1, D), lambda i: (0, 0)),
            pl.BlockSpec((1, 1), lambda i: (0, 0)),
        ],
        out_specs=[
            pl.BlockSpec((512, 1), lambda i: (i, 0)),
            pl.BlockSpec((512, 1), lambda i: (i, 0)),
        ],
        out_shape=[
            jax.ShapeDtypeStruct((NP, 1), jnp.float32),
            jax.ShapeDtypeStruct((NP, 1), jnp.float32),
        ],
    )(agg, nd, b, we_top, we_bot, be)


def _norm_body(h_ref, o_ref):
    deg = h_ref[0, 0][:, 0:1]
    o_ref[0, 0] = jnp.where(deg > 0.0,
                            lax.rsqrt(jnp.maximum(deg, 1.0)),
                            0.0)


def _tc_norms(hist):
    return pl.pallas_call(
        _norm_body,
        grid=(3, 2, 100),
        in_specs=[pl.BlockSpec((1, 1, 512, 16), lambda l, d, i: (l, d, i, 0))],
        out_specs=pl.BlockSpec((1, 1, 512, 1), lambda l, d, i: (l, d, i, 0)),
        out_shape=jax.ShapeDtypeStruct((3, 2, NP, 1), jnp.float32),
    )(hist)


def _hist_body(ei_hbm, out_hbm, idx_v, ones_v, zrow_v, acc_sh):
    c = lax.axis_index("c")
    s = lax.axis_index("s")

    one_hot = jnp.concatenate(
        [jnp.ones((1,), jnp.float32), jnp.zeros((15,), jnp.float32)])
    zeros16 = jnp.zeros((16,), jnp.float32)

    @pl.loop(0, HCHUNK)
    def _(i):
        ones_v[i, :] = one_hot

    @pl.loop(0, 200)
    def _(i):
        zrow_v[i, :] = zeros16

    for layer in range(3):
        for j in range(HROWS_PER_SUB // 200):
            pltpu.sync_copy(
                zrow_v, acc_sh.at[pl.ds(s * HROWS_PER_SUB + j * 200, 200)])
        plsc.subcore_barrier()

        base = s * (E // NSUB)

        @pl.loop(0, E // NSUB, step=HCHUNK)
        def _(off):
            pltpu.sync_copy(ei_hbm.at[layer, c, pl.ds(base + off, HCHUNK)],
                            idx_v)
            pltpu.sync_copy(ones_v, acc_sh.at[idx_v], add=True)

        plsc.subcore_barrier()

        pltpu.sync_copy(
            acc_sh.at[pl.ds(s * HROWS_PER_SUB, HROWS_PER_SUB)],
            out_hbm.at[layer, c, pl.ds(s * HROWS_PER_SUB, HROWS_PER_SUB)])
        plsc.subcore_barrier()


def _sc_hist(ei_all):
    kern = pl.kernel(
        _hist_body,
        out_type=jax.ShapeDtypeStruct((3, 2, NP, 16), jnp.float32),
        mesh=_vmesh,
        scratch_types=[
            pltpu.VMEM((HCHUNK,), jnp.int32),
            pltpu.VMEM((HCHUNK, 16), jnp.float32),
            pltpu.VMEM((200, 16), jnp.float32),
            pltpu.VMEM_SHARED((NP, 16), jnp.float32),
        ],
    )
    return kern(ei_all)


def _agg_body(hw_hbm, src_hbm, dst_hbm, out_hbm,
              s_in, d_in, comp_src, comp_loc, srcq, locq, rows_v, zbuf,
              acc_sh):
    c = lax.axis_index("c")
    s = lax.axis_index("s")

    zeros16f = jnp.zeros((16,), jnp.float32)
    zeros16i = jnp.zeros((16,), jnp.int32)
    dump16 = jnp.full((16,), WN, jnp.int32)

    @pl.loop(0, 50)
    def _(i):
        @pl.loop(0, D, step=16)
        def _(j):
            zbuf[i, pl.ds(j, 16)] = zeros16f

    ebase = s * (E // NSUB)

    for r in range(WPC):
        lo = (c * WPC + r) * WN

        for j in range(8):
            pltpu.sync_copy(zbuf, acc_sh.at[pl.ds(s * 400 + j * 50, 50)])
        plsc.subcore_barrier()

        @pl.loop(0, E // NSUB, step=ECHUNK)
        def _(off):
            pltpu.sync_copy(src_hbm.at[pl.ds(ebase + off, ECHUNK)], s_in)
            pltpu.sync_copy(dst_hbm.at[pl.ds(ebase + off, ECHUNK)], d_in)

            def compact(k, n):
                dd = d_in[pl.ds(k * 16, 16)]
                ss = s_in[pl.ds(k * 16, 16)]
                loc = dd - lo
                m = (loc >= 0) & (loc < WN)
                plsc.store_compressed(comp_src.at[pl.ds(n, 16)], ss, mask=m)
                plsc.store_compressed(comp_loc.at[pl.ds(n, 16)], loc, mask=m)
                return n + jnp.sum(m.astype(jnp.int32))

            n = lax.fori_loop(0, ECHUNK // 16, compact, 0)

            for j in range(GQ // 16):
                comp_src[pl.ds(n + j * 16, 16)] = zeros16i
                comp_loc[pl.ds(n + j * 16, 16)] = dump16
            nq = (n + (GQ - 1)) // GQ

            def fire(q, _):
                for j in range(GQ // 16):
                    srcq[0, pl.ds(j * 16, 16)] = comp_src[
                        pl.ds(q * GQ + j * 16, 16)]
                    locq[0, pl.ds(j * 16, 16)] = comp_loc[
                        pl.ds(q * GQ + j * 16, 16)]
                pltpu.sync_copy(hw_hbm.at[srcq.at[0]], rows_v)
                pltpu.sync_copy(rows_v, acc_sh.at[locq.at[0]], add=True)
                return 0

            lax.fori_loop(0, nq, fire, 0)

        plsc.subcore_barrier()

        pltpu.sync_copy(acc_sh.at[pl.ds(s * 400, 400)],
                        out_hbm.at[pl.ds(lo + s * 400, 400)])
        plsc.subcore_barrier()


def _sc_aggregate(hw, src, dst):
    kern = pl.kernel(
        _agg_body,
        out_type=jax.ShapeDtypeStruct((NP, D), jnp.float32),
        mesh=_vmesh,
        scratch_types=[
            pltpu.VMEM((ECHUNK,), jnp.int32),
            pltpu.VMEM((ECHUNK,), jnp.int32),
            pltpu.VMEM((ECHUNK + 2 * GQ,), jnp.int32),
            pltpu.VMEM((ECHUNK + 2 * GQ,), jnp.int32),
            pltpu.VMEM((1, GQ), jnp.int32),
            pltpu.VMEM((1, GQ), jnp.int32),
            pltpu.VMEM((GQ, D), jnp.float32),
            pltpu.VMEM((50, D), jnp.float32),
            pltpu.VMEM_SHARED((ACC_ROWS, D), jnp.float32),
        ],
    )
    return kern(hw, src, dst)


def _pred_body(u_hbm, v_hbm, ps_hbm, pd_hbm, out_hbm,
               u_v, v_v, ps_v, pd_v, o_v):
    c = lax.axis_index("c")
    s = lax.axis_index("s")
    w = s * NSC + c

    pltpu.sync_copy(u_hbm, u_v)
    pltpu.sync_copy(v_hbm, v_v)

    zeros16 = jnp.zeros((16,), jnp.int32)
    base = w * (E // (NSC * NSUB))

    @pl.loop(0, E // (NSC * NSUB), step=HCHUNK)
    def _(off):
        pltpu.sync_copy(ps_hbm.at[pl.ds(base + off, HCHUNK)], ps_v)
        pltpu.sync_copy(pd_hbm.at[pl.ds(base + off, HCHUNK)], pd_v)

        @pl.loop(0, HCHUNK, step=16)
        def _(k):
            pi = ps_v[pl.ds(k, 16)]
            di = pd_v[pl.ds(k, 16)]
            us = plsc.load_gather(u_v, [pi, zeros16])
            vs = plsc.load_gather(v_v, [di, zeros16])
            o_v[pl.ds(k, 16)] = us + vs

        pltpu.sync_copy(o_v, out_hbm.at[pl.ds(base + off, HCHUNK)])


def _sc_pred(u, v, ps, pd):
    kern = pl.kernel(
        _pred_body,
        out_type=jax.ShapeDtypeStruct((E,), jnp.float32),
        mesh=_vmesh,
        scratch_types=[
            pltpu.VMEM((NP, 1), jnp.float32),
            pltpu.VMEM((NP, 1), jnp.float32),
            pltpu.VMEM((HCHUNK,), jnp.int32),
            pltpu.VMEM((HCHUNK,), jnp.int32),
            pltpu.VMEM((HCHUNK,), jnp.float32),
        ],
    )
    return kern(u, v, ps, pd)


def kernel(x, edge_index0, edge_index1, edge_index2, pred_edge_index,
           Wr, br, W1, b1, W2, b2, W3, b3, We, be):
    ei_all = jnp.stack([edge_index0.astype(jnp.int32),
                        edge_index1.astype(jnp.int32),
                        edge_index2.astype(jnp.int32)])
    pei = pred_edge_index.astype(jnp.int32)

    hist = _sc_hist(ei_all)
    norms = _tc_norms(hist)

    h0 = _tc_reducer(x, Wr, br.reshape(1, D))

    hw1 = _tc_scale_matmul(h0, norms[0, 0], W1)
    agg1 = _sc_aggregate(hw1, ei_all[0, 0], ei_all[0, 1])

    hw2 = _tc_post_matmul(agg1, norms[0, 1], b1.reshape(1, D), norms[1, 0], W2)
    agg2 = _sc_aggregate(hw2, ei_all[1, 0], ei_all[1, 1])

    hw3 = _tc_post_matmul(agg2, norms[1, 1], b2.reshape(1, D), norms[2, 0], W3)
    agg3 = _sc_aggregate(hw3, ei_all[2, 0], ei_all[2, 1])

    u, v = _tc_uv(agg3, norms[2, 1], b3.reshape(1, D),
                  We[:D, 0].reshape(1, D), We[D:, 0].reshape(1, D),
                  be.reshape(1, 1))

    score = _sc_pred(u, v, pei[0], pei[1])
    return score.reshape(E, 1)


# speedup vs baseline: 1.7544x; 1.7544x over previous
import functools

import jax
import jax.numpy as jnp
from jax import lax
from jax.experimental import pallas as pl
from jax.experimental.pallas import tpu as pltpu
from jax.experimental.pallas import tpu_sc as plsc

N = 50000
NP = 57344
E = 1600000
D_IN = 512
D = 256

NSC = 2
NSUB = 16
LANES = 16

WN = 4096
WPC = 7
ACC_ROWS = WN + 8
ECHUNK = 4000
GQ = 64

HCHUNK = 2000
HROWS_PER_SUB = NP // NSUB

_vmesh = plsc.VectorSubcoreMesh(core_axis_name="c", subcore_axis_name="s")
_sc_params = pltpu.CompilerParams(use_tc_tiling_on_sc=False,
                                  needs_layout_passes=False)


def _reducer_body(x_ref, w_ref, b_ref, o_ref):
    o_ref[...] = (
        jnp.dot(x_ref[...], w_ref[...], preferred_element_type=jnp.float32)
        + b_ref[...]
    )


def _tc_reducer(x, wr, br):
    return pl.pallas_call(
        _reducer_body,
        grid=(125,),
        in_specs=[
            pl.BlockSpec((400, D_IN), lambda i: (i, 0)),
            pl.BlockSpec((D_IN, D), lambda i: (0, 0)),
            pl.BlockSpec((1, D), lambda i: (0, 0)),
        ],
        out_specs=pl.BlockSpec((400, D), lambda i: (i, 0)),
        out_shape=jax.ShapeDtypeStruct((NP, D), jnp.float32),
    )(x, wr, br)


def _scale_mm_body(h_ref, ns_ref, w_ref, o_ref):
    o_ref[...] = jnp.dot(
        h_ref[...] * ns_ref[...], w_ref[...],
        preferred_element_type=jnp.float32)


def _tc_scale_matmul(h, ns, w):
    return pl.pallas_call(
        _scale_mm_body,
        grid=(NP // 512,),
        in_specs=[
            pl.BlockSpec((512, D), lambda i: (i, 0)),
            pl.BlockSpec((512, 1), lambda i: (i, 0)),
            pl.BlockSpec((D, D), lambda i: (0, 0)),
        ],
        out_specs=pl.BlockSpec((512, D), lambda i: (i, 0)),
        out_shape=jax.ShapeDtypeStruct((NP, D), jnp.float32),
    )(h, ns, w)


def _post_mm_body(a_ref, nd_ref, b_ref, ns_ref, w_ref, o_ref):
    z = jax.nn.relu(a_ref[...] * nd_ref[...] + b_ref[...])
    o_ref[...] = jnp.dot(z * ns_ref[...], w_ref[...],
                         preferred_element_type=jnp.float32)


def _tc_post_matmul(agg, nd, b, ns, w):
    return pl.pallas_call(
        _post_mm_body,
        grid=(NP // 512,),
        in_specs=[
            pl.BlockSpec((512, D), lambda i: (i, 0)),
            pl.BlockSpec((512, 1), lambda i: (i, 0)),
            pl.BlockSpec((1, D), lambda i: (0, 0)),
            pl.BlockSpec((512, 1), lambda i: (i, 0)),
            pl.BlockSpec((D, D), lambda i: (0, 0)),
        ],
        out_specs=pl.BlockSpec((512, D), lambda i: (i, 0)),
        out_shape=jax.ShapeDtypeStruct((NP, D), jnp.float32),
    )(agg, nd, b, ns, w)


def _uv_body(a_ref, nd_ref, b_ref, wt_ref, wb_ref, be_ref, u_ref, v_ref):
    z = jax.nn.relu(a_ref[...] * nd_ref[...] + b_ref[...])
    u_ref[...] = jnp.sum(z * wt_ref[...], axis=1, keepdims=True) + be_ref[0, 0]
    v_ref[...] = jnp.sum(z * wb_ref[...], axis=1, keepdims=True)


def _tc_uv(agg, nd, b, we_top, we_bot, be):
    return pl.pallas_call(
        _uv_body,
        grid=(NP // 512,),
        in_specs=[
            pl.BlockSpec((512, D), lambda i: (i, 0)),
            pl.BlockSpec((512, 1), lambda i: (i, 0)),
            pl.BlockSpec((1, D), lambda i: (0, 0)),
            pl.BlockSpec((1, D), lambda i: (0, 0)),
            pl.BlockSpec((1, D), lambda i: (0, 0)),
            pl.BlockSpec((1, 1), lambda i: (0, 0)),
        ],
        out_specs=[
            pl.BlockSpec((512, 1), lambda i: (i, 0)),
            pl.BlockSpec((512, 1), lambda i: (i, 0)),
        ],
        out_shape=[
            jax.ShapeDtypeStruct((NP, 1), jnp.float32),
            jax.ShapeDtypeStruct((NP, 1), jnp.float32),
        ],
    )(agg, nd, b, we_top, we_bot, be)


def _norm_body(h_ref, o_ref):
    deg = h_ref[0, 0][:, 0:1]
    o_ref[0, 0] = jnp.where(deg > 0.0,
                            lax.rsqrt(jnp.maximum(deg, 1.0)),
                            0.0)


def _tc_norms(hist):
    return pl.pallas_call(
        _norm_body,
        grid=(3, 2, NP // 512),
        in_specs=[pl.BlockSpec((1, 1, 512, 16), lambda l, d, i: (l, d, i, 0))],
        out_specs=pl.BlockSpec((1, 1, 512, 1), lambda l, d, i: (l, d, i, 0)),
        out_shape=jax.ShapeDtypeStruct((3, 2, NP, 1), jnp.float32),
    )(hist)


def _hist_body(ei_hbm, out_hbm, idx_v, ones_v, zrow_v, acc_sh):
    c = lax.axis_index("c")
    s = lax.axis_index("s")

    one_hot = jnp.where(lax.iota(jnp.int32, 16) == 0, 1.0, 0.0)
    zeros16 = jnp.zeros((16,), jnp.float32)

    @pl.loop(0, HCHUNK)
    def _(i):
        ones_v[i, :] = one_hot

    @pl.loop(0, 224)
    def _(i):
        zrow_v[i, :] = zeros16

    for layer in range(3):
        for j in range(HROWS_PER_SUB // 224):
            pltpu.sync_copy(
                zrow_v, acc_sh.at[pl.ds(s * HROWS_PER_SUB + j * 224, 224)])
        plsc.subcore_barrier()

        base = (layer * 2 + c) * E + s * (E // NSUB)

        @pl.loop(0, E // NSUB, step=HCHUNK)
        def _(off):
            pltpu.sync_copy(ei_hbm.at[pl.ds(base + off, HCHUNK)], idx_v)
            pltpu.sync_copy(ones_v, acc_sh.at[idx_v], add=True)

        plsc.subcore_barrier()

        pltpu.sync_copy(
            acc_sh.at[pl.ds(s * HROWS_PER_SUB, HROWS_PER_SUB)],
            out_hbm.at[layer, c, pl.ds(s * HROWS_PER_SUB, HROWS_PER_SUB)])
        plsc.subcore_barrier()


def _sc_hist(ei_flat):
    kern = pl.kernel(
        _hist_body,
        out_type=jax.ShapeDtypeStruct((3, 2, NP, 16), jnp.float32),
        mesh=_vmesh,
        compiler_params=_sc_params,
        scratch_types=[
            pltpu.VMEM((HCHUNK,), jnp.int32),
            pltpu.VMEM((HCHUNK, 16), jnp.float32),
            pltpu.VMEM((224, 16), jnp.float32),
            pltpu.VMEM_SHARED((NP, 16), jnp.float32),
        ],
    )
    return kern(ei_flat)


def _agg_body(hw_hbm, src_hbm, dst_hbm, out_hbm,
              s_in, d_in, comp_src, comp_loc, srcq, locq, rows_v, zbuf,
              acc_sh):
    c = lax.axis_index("c")
    s = lax.axis_index("s")

    zeros16f = jnp.zeros((16,), jnp.float32)
    zeros16i = jnp.zeros((16,), jnp.int32)
    dump16 = jnp.full((16,), WN, jnp.int32)

    @pl.loop(0, 64)
    def _(i):
        @pl.loop(0, D, step=16)
        def _(j):
            zbuf[i, pl.ds(j, 16)] = zeros16f

    ebase = s * (E // NSUB)

    for r in range(WPC):
        lo = (c * WPC + r) * WN

        for j in range(4):
            pltpu.sync_copy(zbuf, acc_sh.at[pl.ds(s * 256 + j * 64, 64)])
        plsc.subcore_barrier()

        @pl.loop(0, E // NSUB, step=ECHUNK)
        def _(off):
            pltpu.sync_copy(src_hbm.at[pl.ds(ebase + off, ECHUNK)], s_in)
            pltpu.sync_copy(dst_hbm.at[pl.ds(ebase + off, ECHUNK)], d_in)

            def compact(k, n):
                dd = d_in[pl.ds(k * 16, 16)]
                ss = s_in[pl.ds(k * 16, 16)]
                loc = dd - lo
                m = (loc >= 0) & (loc < WN)
                plsc.store_compressed(comp_src.at[pl.ds(n, 16)], ss, mask=m)
                plsc.store_compressed(comp_loc.at[pl.ds(n, 16)], loc, mask=m)
                return n + jnp.sum(m.astype(jnp.int32))

            n = lax.fori_loop(0, ECHUNK // 16, compact, 0)

            for j in range(GQ // 16):
                comp_src[pl.ds(n + j * 16, 16)] = zeros16i
                comp_loc[pl.ds(n + j * 16, 16)] = dump16
            nq = (n + (GQ - 1)) // GQ

            def fire(q, _):
                for j in range(GQ // 16):
                    srcq[0, pl.ds(j * 16, 16)] = comp_src[
                        pl.ds(q * GQ + j * 16, 16)]
                    locq[0, pl.ds(j * 16, 16)] = comp_loc[
                        pl.ds(q * GQ + j * 16, 16)]
                pltpu.sync_copy(hw_hbm.at[srcq.at[0]], rows_v)
                pltpu.sync_copy(rows_v, acc_sh.at[locq.at[0]], add=True)
                return 0

            lax.fori_loop(0, nq, fire, 0)

        plsc.subcore_barrier()

        pltpu.sync_copy(acc_sh.at[pl.ds(s * 256, 256)],
                        out_hbm.at[pl.ds(lo + s * 256, 256)])
        plsc.subcore_barrier()


def _sc_aggregate(hw, src, dst):
    kern = pl.kernel(
        _agg_body,
        out_type=jax.ShapeDtypeStruct((NP, D), jnp.float32),
        mesh=_vmesh,
        compiler_params=_sc_params,
        scratch_types=[
            pltpu.VMEM((ECHUNK,), jnp.int32),
            pltpu.VMEM((ECHUNK,), jnp.int32),
            pltpu.VMEM((ECHUNK + 2 * GQ,), jnp.int32),
            pltpu.VMEM((ECHUNK + 2 * GQ,), jnp.int32),
            pltpu.VMEM((1, GQ), jnp.int32),
            pltpu.VMEM((1, GQ), jnp.int32),
            pltpu.VMEM((GQ, D), jnp.float32),
            pltpu.VMEM((64, D), jnp.float32),
            pltpu.VMEM_SHARED((ACC_ROWS, D), jnp.float32),
        ],
    )
    return kern(hw, src, dst)


def _pred_body(u_hbm, v_hbm, ps_hbm, pd_hbm, out_hbm,
               u_v, v_v, ps_v, pd_v, o_v):
    c = lax.axis_index("c")
    s = lax.axis_index("s")
    w = s * NSC + c

    pltpu.sync_copy(u_hbm, u_v)
    pltpu.sync_copy(v_hbm, v_v)

    base = w * (E // (NSC * NSUB))

    @pl.loop(0, E // (NSC * NSUB), step=HCHUNK)
    def _(off):
        pltpu.sync_copy(ps_hbm.at[pl.ds(base + off, HCHUNK)], ps_v)
        pltpu.sync_copy(pd_hbm.at[pl.ds(base + off, HCHUNK)], pd_v)

        @pl.loop(0, HCHUNK, step=16)
        def _(k):
            pi = ps_v[pl.ds(k, 16)]
            di = pd_v[pl.ds(k, 16)]
            us = plsc.load_gather(u_v, [pi >> 7, pi & 127])
            vs = plsc.load_gather(v_v, [di >> 7, di & 127])
            o_v[pl.ds(k, 16)] = us + vs

        pltpu.sync_copy(o_v, out_hbm.at[pl.ds(base + off, HCHUNK)])


def _sc_pred(u, v, ps, pd):
    kern = pl.kernel(
        _pred_body,
        out_type=jax.ShapeDtypeStruct((E,), jnp.float32),
        mesh=_vmesh,
        compiler_params=_sc_params,
        scratch_types=[
            pltpu.VMEM((NP // 128, 128), jnp.float32),
            pltpu.VMEM((NP // 128, 128), jnp.float32),
            pltpu.VMEM((HCHUNK,), jnp.int32),
            pltpu.VMEM((HCHUNK,), jnp.int32),
            pltpu.VMEM((HCHUNK,), jnp.float32),
        ],
    )
    return kern(u, v, ps, pd)


def kernel(x, edge_index0, edge_index1, edge_index2, pred_edge_index,
           Wr, br, W1, b1, W2, b2, W3, b3, We, be):
    ei_all = jnp.stack([edge_index0.astype(jnp.int32),
                        edge_index1.astype(jnp.int32),
                        edge_index2.astype(jnp.int32)])
    pei = pred_edge_index.astype(jnp.int32)

    hist = _sc_hist(ei_all.reshape(6 * E))
    norms = _tc_norms(hist)

    h0 = _tc_reducer(x, Wr, br.reshape(1, D))

    hw1 = _tc_scale_matmul(h0, norms[0, 0], W1)
    agg1 = _sc_aggregate(hw1, ei_all[0, 0], ei_all[0, 1])

    hw2 = _tc_post_matmul(agg1, norms[0, 1], b1.reshape(1, D), norms[1, 0], W2)
    agg2 = _sc_aggregate(hw2, ei_all[1, 0], ei_all[1, 1])

    hw3 = _tc_post_matmul(agg2, norms[1, 1], b2.reshape(1, D), norms[2, 0], W3)
    agg3 = _sc_aggregate(hw3, ei_all[2, 0], ei_all[2, 1])

    u, v = _tc_uv(agg3, norms[2, 1], b3.reshape(1, D),
                  We[:D, 0].reshape(1, D), We[D:, 0].reshape(1, D),
                  be.reshape(1, 1))

    score = _sc_pred(u.reshape(NP // 128, 128), v.reshape(NP // 128, 128),
                     pei[0], pei[1])
    return score.reshape(E, 1)


# speedup vs baseline: 1.7725x; 1.0104x over previous
import functools

import jax
import jax.numpy as jnp
from jax import lax
from jax.experimental import pallas as pl
from jax.experimental.pallas import tpu as pltpu
from jax.experimental.pallas import tpu_sc as plsc

N = 50000
NP = 57344
E = 1600000
D_IN = 512
D = 256

NSC = 2
NSUB = 16
LANES = 16

WN = 4096
WPC = 7
ACC_ROWS = WN + 8
ECHUNK = 4000
GQ = 64

HCHUNK = 2000
HROWS_PER_SUB = NP // NSUB

_vmesh = plsc.VectorSubcoreMesh(core_axis_name="c", subcore_axis_name="s")
_sc_params = pltpu.CompilerParams(use_tc_tiling_on_sc=False,
                                  needs_layout_passes=False)


def _reducer_body(x_ref, w_ref, b_ref, o_ref):
    o_ref[...] = (
        jnp.dot(x_ref[...], w_ref[...], preferred_element_type=jnp.float32)
        + b_ref[...]
    )


def _tc_reducer(x, wr, br):
    return pl.pallas_call(
        _reducer_body,
        grid=(125,),
        in_specs=[
            pl.BlockSpec((400, D_IN), lambda i: (i, 0)),
            pl.BlockSpec((D_IN, D), lambda i: (0, 0)),
            pl.BlockSpec((1, D), lambda i: (0, 0)),
        ],
        out_specs=pl.BlockSpec((400, D), lambda i: (i, 0)),
        out_shape=jax.ShapeDtypeStruct((NP, D), jnp.float32),
    )(x, wr, br)


def _scale_mm_body(h_ref, ns_ref, w_ref, o_ref):
    o_ref[...] = jnp.dot(
        h_ref[...] * ns_ref[...], w_ref[...],
        preferred_element_type=jnp.float32)


def _tc_scale_matmul(h, ns, w):
    return pl.pallas_call(
        _scale_mm_body,
        grid=(NP // 512,),
        in_specs=[
            pl.BlockSpec((512, D), lambda i: (i, 0)),
            pl.BlockSpec((512, 1), lambda i: (i, 0)),
            pl.BlockSpec((D, D), lambda i: (0, 0)),
        ],
        out_specs=pl.BlockSpec((512, D), lambda i: (i, 0)),
        out_shape=jax.ShapeDtypeStruct((NP, D), jnp.float32),
    )(h, ns, w)


def _post_mm_body(a_ref, nd_ref, b_ref, ns_ref, w_ref, o_ref):
    z = jax.nn.relu(a_ref[...] * nd_ref[...] + b_ref[...])
    o_ref[...] = jnp.dot(z * ns_ref[...], w_ref[...],
                         preferred_element_type=jnp.float32)


def _tc_post_matmul(agg, nd, b, ns, w):
    return pl.pallas_call(
        _post_mm_body,
        grid=(NP // 512,),
        in_specs=[
            pl.BlockSpec((512, D), lambda i: (i, 0)),
            pl.BlockSpec((512, 1), lambda i: (i, 0)),
            pl.BlockSpec((1, D), lambda i: (0, 0)),
            pl.BlockSpec((512, 1), lambda i: (i, 0)),
            pl.BlockSpec((D, D), lambda i: (0, 0)),
        ],
        out_specs=pl.BlockSpec((512, D), lambda i: (i, 0)),
        out_shape=jax.ShapeDtypeStruct((NP, D), jnp.float32),
    )(agg, nd, b, ns, w)


def _uv_body(a_ref, nd_ref, b_ref, wt_ref, wb_ref, be_ref, u_ref, v_ref):
    z = jax.nn.relu(a_ref[...] * nd_ref[...] + b_ref[...])
    u_ref[...] = jnp.sum(z * wt_ref[...], axis=1, keepdims=True) + be_ref[0, 0]
    v_ref[...] = jnp.sum(z * wb_ref[...], axis=1, keepdims=True)


def _tc_uv(agg, nd, b, we_top, we_bot, be):
    return pl.pallas_call(
        _uv_body,
        grid=(NP // 512,),
        in_specs=[
            pl.BlockSpec((512, D), lambda i: (i, 0)),
            pl.BlockSpec((512, 1), lambda i: (i, 0)),
            pl.BlockSpec((1, D), lambda i: (0, 0)),
            pl.BlockSpec((1, D), lambda i: (0, 0)),
            pl.BlockSpec((1, D), lambda i: (0, 0)),
            pl.BlockSpec((1, 1), lambda i: (0, 0)),
        ],
        out_specs=[
            pl.BlockSpec((512, 1), lambda i: (i, 0)),
            pl.BlockSpec((512, 1), lambda i: (i, 0)),
        ],
        out_shape=[
            jax.ShapeDtypeStruct((NP, 1), jnp.float32),
            jax.ShapeDtypeStruct((NP, 1), jnp.float32),
        ],
    )(agg, nd, b, we_top, we_bot, be)


def _norm_body(h_ref, o_ref):
    deg = h_ref[0, 0][:, 0:1]
    o_ref[0, 0] = jnp.where(deg > 0.0,
                            lax.rsqrt(jnp.maximum(deg, 1.0)),
                            0.0)


def _tc_norms(hist):
    return pl.pallas_call(
        _norm_body,
        grid=(3, 2, NP // 512),
        in_specs=[pl.BlockSpec((1, 1, 512, 16), lambda l, d, i: (l, d, i, 0))],
        out_specs=pl.BlockSpec((1, 1, 512, 1), lambda l, d, i: (l, d, i, 0)),
        out_shape=jax.ShapeDtypeStruct((3, 2, NP, 1), jnp.float32),
    )(hist)


def _hist_body(ei_hbm, out_hbm, idx_v, ones_v, zrow_v, acc_sh):
    c = lax.axis_index("c")
    s = lax.axis_index("s")

    one_hot = jnp.where(lax.iota(jnp.int32, 16) == 0, 1.0, 0.0)
    zeros16 = jnp.zeros((16,), jnp.float32)

    @pl.loop(0, HCHUNK)
    def _(i):
        ones_v[i, :] = one_hot

    @pl.loop(0, 224)
    def _(i):
        zrow_v[i, :] = zeros16

    for layer in range(3):
        for j in range(HROWS_PER_SUB // 224):
            pltpu.sync_copy(
                zrow_v, acc_sh.at[pl.ds(s * HROWS_PER_SUB + j * 224, 224)])
        plsc.subcore_barrier()

        base = (layer * 2 + c) * E + s * (E // NSUB)

        @pl.loop(0, E // NSUB, step=HCHUNK)
        def _(off):
            pltpu.sync_copy(ei_hbm.at[pl.ds(base + off, HCHUNK)], idx_v)
            pltpu.sync_copy(ones_v, acc_sh.at[idx_v], add=True)

        plsc.subcore_barrier()

        pltpu.sync_copy(
            acc_sh.at[pl.ds(s * HROWS_PER_SUB, HROWS_PER_SUB)],
            out_hbm.at[layer, c, pl.ds(s * HROWS_PER_SUB, HROWS_PER_SUB)])
        plsc.subcore_barrier()


def _sc_hist(ei_flat):
    kern = pl.kernel(
        _hist_body,
        out_type=jax.ShapeDtypeStruct((3, 2, NP, 16), jnp.float32),
        mesh=_vmesh,
        compiler_params=_sc_params,
        scratch_types=[
            pltpu.VMEM((HCHUNK,), jnp.int32),
            pltpu.VMEM((HCHUNK, 16), jnp.float32),
            pltpu.VMEM((224, 16), jnp.float32),
            pltpu.VMEM_SHARED((NP, 16), jnp.float32),
        ],
    )
    return kern(ei_flat)


def _agg_body(hw_hbm, src_hbm, dst_hbm, out_hbm,
              s_in, d_in, comp_src, comp_loc, srcq, locq, rows_v, zbuf,
              acc_sh, gsem, ssem):
    c = lax.axis_index("c")
    s = lax.axis_index("s")

    zeros16f = jnp.zeros((16,), jnp.float32)
    zeros16i = jnp.zeros((16,), jnp.int32)
    dump16 = jnp.full((16,), WN, jnp.int32)

    @pl.loop(0, 32)
    def _(i):
        @pl.loop(0, D, step=16)
        def _(j):
            zbuf[i, pl.ds(j, 16)] = zeros16f

    ebase = s * (E // NSUB)

    for r in range(WPC):
        lo = (c * WPC + r) * WN

        for j in range(8):
            pltpu.sync_copy(zbuf, acc_sh.at[pl.ds(s * 256 + j * 32, 32)])
        plsc.subcore_barrier()

        @pl.loop(0, E // NSUB, step=ECHUNK)
        def _(off):
            pltpu.sync_copy(src_hbm.at[pl.ds(ebase + off, ECHUNK)], s_in)
            pltpu.sync_copy(dst_hbm.at[pl.ds(ebase + off, ECHUNK)], d_in)

            def compact(k, n):
                dd = d_in[pl.ds(k * 16, 16)]
                ss = s_in[pl.ds(k * 16, 16)]
                loc = dd - lo
                m = (loc >= 0) & (loc < WN)
                plsc.store_compressed(comp_src.at[pl.ds(n, 16)], ss, mask=m)
                plsc.store_compressed(comp_loc.at[pl.ds(n, 16)], loc, mask=m)
                return n + jnp.sum(m.astype(jnp.int32))

            n = lax.fori_loop(0, ECHUNK // 16, compact, 0)

            for j in range(GQ // 16):
                comp_src[pl.ds(n + j * 16, 16)] = zeros16i
                comp_loc[pl.ds(n + j * 16, 16)] = dump16
            nq = (n + (GQ - 1)) // GQ

            def gather_start(q):
                p = q & 1
                for j in range(GQ // 16):
                    srcq[p, pl.ds(j * 16, 16)] = comp_src[
                        pl.ds(q * GQ + j * 16, 16)]
                    locq[p, pl.ds(j * 16, 16)] = comp_loc[
                        pl.ds(q * GQ + j * 16, 16)]
                pltpu.async_copy(hw_hbm.at[srcq.at[p]], rows_v.at[p],
                                 gsem.at[p])

            def scat_start(q):
                p = q & 1
                pltpu.make_async_copy(hw_hbm.at[srcq.at[p]], rows_v.at[p],
                                      gsem.at[p]).wait()
                pltpu.async_copy(rows_v.at[p], acc_sh.at[locq.at[p]],
                                 ssem.at[p], add=True)

            def scat_wait(q):
                p = q & 1
                pltpu.make_async_copy(rows_v.at[p], acc_sh.at[locq.at[p]],
                                      ssem.at[p]).wait()

            def fire(q, _):
                pl.when(q >= 2)(lambda: scat_wait(q))
                gather_start(q)
                pl.when(q >= 1)(lambda: scat_start(q - 1))
                return 0

            lax.fori_loop(0, nq, fire, 0)
            pl.when(nq >= 1)(lambda: scat_start(nq - 1))
            pl.when(nq >= 2)(lambda: scat_wait(nq - 2))
            pl.when(nq >= 1)(lambda: scat_wait(nq - 1))

        plsc.subcore_barrier()

        pltpu.sync_copy(acc_sh.at[pl.ds(s * 256, 256)],
                        out_hbm.at[pl.ds(lo + s * 256, 256)])
        plsc.subcore_barrier()


def _sc_aggregate(hw, src, dst):
    kern = pl.kernel(
        _agg_body,
        out_type=jax.ShapeDtypeStruct((NP, D), jnp.float32),
        mesh=_vmesh,
        compiler_params=_sc_params,
        scratch_types=[
            pltpu.VMEM((ECHUNK,), jnp.int32),
            pltpu.VMEM((ECHUNK,), jnp.int32),
            pltpu.VMEM((ECHUNK + 2 * GQ,), jnp.int32),
            pltpu.VMEM((ECHUNK + 2 * GQ,), jnp.int32),
            pltpu.VMEM((2, GQ), jnp.int32),
            pltpu.VMEM((2, GQ), jnp.int32),
            pltpu.VMEM((2, GQ, D), jnp.float32),
            pltpu.VMEM((32, D), jnp.float32),
            pltpu.VMEM_SHARED((ACC_ROWS, D), jnp.float32),
            pltpu.SemaphoreType.DMA((2,)),
            pltpu.SemaphoreType.DMA((2,)),
        ],
    )
    return kern(hw, src, dst)


def _pred_body(u_hbm, v_hbm, ps_hbm, pd_hbm, out_hbm,
               u_v, v_v, ps_v, pd_v, o_v):
    c = lax.axis_index("c")
    s = lax.axis_index("s")
    w = s * NSC + c

    pltpu.sync_copy(u_hbm, u_v)
    pltpu.sync_copy(v_hbm, v_v)

    base = w * (E // (NSC * NSUB))

    @pl.loop(0, E // (NSC * NSUB), step=HCHUNK)
    def _(off):
        pltpu.sync_copy(ps_hbm.at[pl.ds(base + off, HCHUNK)], ps_v)
        pltpu.sync_copy(pd_hbm.at[pl.ds(base + off, HCHUNK)], pd_v)

        @pl.loop(0, HCHUNK, step=16)
        def _(k):
            pi = ps_v[pl.ds(k, 16)]
            di = pd_v[pl.ds(k, 16)]
            us = plsc.load_gather(u_v, [pi >> 7, pi & 127])
            vs = plsc.load_gather(v_v, [di >> 7, di & 127])
            o_v[pl.ds(k, 16)] = us + vs

        pltpu.sync_copy(o_v, out_hbm.at[pl.ds(base + off, HCHUNK)])


def _sc_pred(u, v, ps, pd):
    kern = pl.kernel(
        _pred_body,
        out_type=jax.ShapeDtypeStruct((E,), jnp.float32),
        mesh=_vmesh,
        compiler_params=_sc_params,
        scratch_types=[
            pltpu.VMEM((NP // 128, 128), jnp.float32),
            pltpu.VMEM((NP // 128, 128), jnp.float32),
            pltpu.VMEM((HCHUNK,), jnp.int32),
            pltpu.VMEM((HCHUNK,), jnp.int32),
            pltpu.VMEM((HCHUNK,), jnp.float32),
        ],
    )
    return kern(u, v, ps, pd)


def kernel(x, edge_index0, edge_index1, edge_index2, pred_edge_index,
           Wr, br, W1, b1, W2, b2, W3, b3, We, be):
    ei_all = jnp.stack([edge_index0.astype(jnp.int32),
                        edge_index1.astype(jnp.int32),
                        edge_index2.astype(jnp.int32)])
    pei = pred_edge_index.astype(jnp.int32)

    hist = _sc_hist(ei_all.reshape(6 * E))
    norms = _tc_norms(hist)

    h0 = _tc_reducer(x, Wr, br.reshape(1, D))

    hw1 = _tc_scale_matmul(h0, norms[0, 0], W1)
    agg1 = _sc_aggregate(hw1, ei_all[0, 0], ei_all[0, 1])

    hw2 = _tc_post_matmul(agg1, norms[0, 1], b1.reshape(1, D), norms[1, 0], W2)
    agg2 = _sc_aggregate(hw2, ei_all[1, 0], ei_all[1, 1])

    hw3 = _tc_post_matmul(agg2, norms[1, 1], b2.reshape(1, D), norms[2, 0], W3)
    agg3 = _sc_aggregate(hw3, ei_all[2, 0], ei_all[2, 1])

    u, v = _tc_uv(agg3, norms[2, 1], b3.reshape(1, D),
                  We[:D, 0].reshape(1, D), We[D:, 0].reshape(1, D),
                  be.reshape(1, 1))

    score = _sc_pred(u.reshape(NP // 128, 128), v.reshape(NP // 128, 128),
                     pei[0], pei[1])
    return score.reshape(E, 1)


# speedup vs baseline: 5.6537x; 3.1896x over previous
import functools

import jax
import jax.numpy as jnp
from jax import lax
from jax.experimental import pallas as pl
from jax.experimental.pallas import tpu as pltpu
from jax.experimental.pallas import tpu_sc as plsc

N = 50000
NP = 57344
E = 1600000
D_IN = 512
D = 256

NSC = 2
NSUB = 16
LANES = 16

WN = 4096
WPC = 7
ACC_ROWS = WN + 8
ECHUNK = 2000
GQ = 64

HCHUNK = 2000
HROWS_PER_SUB = NP // NSUB

_vmesh = plsc.VectorSubcoreMesh(core_axis_name="c", subcore_axis_name="s")
_sc_params = pltpu.CompilerParams(use_tc_tiling_on_sc=False,
                                  needs_layout_passes=False)


def _reducer_body(x_ref, w_ref, b_ref, o_ref):
    o_ref[...] = (
        jnp.dot(x_ref[...], w_ref[...], preferred_element_type=jnp.float32)
        + b_ref[...]
    )


def _tc_reducer(x, wr, br):
    return pl.pallas_call(
        _reducer_body,
        grid=(125,),
        in_specs=[
            pl.BlockSpec((400, D_IN), lambda i: (i, 0)),
            pl.BlockSpec((D_IN, D), lambda i: (0, 0)),
            pl.BlockSpec((1, D), lambda i: (0, 0)),
        ],
        out_specs=pl.BlockSpec((400, D), lambda i: (i, 0)),
        out_shape=jax.ShapeDtypeStruct((NP, D), jnp.float32),
    )(x, wr, br)


def _scale_mm_body(h_ref, ns_ref, w_ref, o_ref):
    o_ref[...] = jnp.dot(
        h_ref[...] * ns_ref[...], w_ref[...],
        preferred_element_type=jnp.float32)


def _tc_scale_matmul(h, ns, w):
    return pl.pallas_call(
        _scale_mm_body,
        grid=(NP // 512,),
        in_specs=[
            pl.BlockSpec((512, D), lambda i: (i, 0)),
            pl.BlockSpec((512, 1), lambda i: (i, 0)),
            pl.BlockSpec((D, D), lambda i: (0, 0)),
        ],
        out_specs=pl.BlockSpec((512, D), lambda i: (i, 0)),
        out_shape=jax.ShapeDtypeStruct((NP, D), jnp.float32),
    )(h, ns, w)


def _post_mm_body(a_ref, nd_ref, b_ref, ns_ref, w_ref, o_ref):
    z = jax.nn.relu(a_ref[...] * nd_ref[...] + b_ref[...])
    o_ref[...] = jnp.dot(z * ns_ref[...], w_ref[...],
                         preferred_element_type=jnp.float32)


def _tc_post_matmul(agg, nd, b, ns, w):
    return pl.pallas_call(
        _post_mm_body,
        grid=(NP // 512,),
        in_specs=[
            pl.BlockSpec((512, D), lambda i: (i, 0)),
            pl.BlockSpec((512, 1), lambda i: (i, 0)),
            pl.BlockSpec((1, D), lambda i: (0, 0)),
            pl.BlockSpec((512, 1), lambda i: (i, 0)),
            pl.BlockSpec((D, D), lambda i: (0, 0)),
        ],
        out_specs=pl.BlockSpec((512, D), lambda i: (i, 0)),
        out_shape=jax.ShapeDtypeStruct((NP, D), jnp.float32),
    )(agg, nd, b, ns, w)


def _uv_body(a_ref, nd_ref, b_ref, wt_ref, wb_ref, be_ref, u_ref, v_ref):
    z = jax.nn.relu(a_ref[...] * nd_ref[...] + b_ref[...])
    u_ref[...] = jnp.sum(z * wt_ref[...], axis=1, keepdims=True) + be_ref[0, 0]
    v_ref[...] = jnp.sum(z * wb_ref[...], axis=1, keepdims=True)


def _tc_uv(agg, nd, b, we_top, we_bot, be):
    return pl.pallas_call(
        _uv_body,
        grid=(NP // 512,),
        in_specs=[
            pl.BlockSpec((512, D), lambda i: (i, 0)),
            pl.BlockSpec((512, 1), lambda i: (i, 0)),
            pl.BlockSpec((1, D), lambda i: (0, 0)),
            pl.BlockSpec((1, D), lambda i: (0, 0)),
            pl.BlockSpec((1, D), lambda i: (0, 0)),
            pl.BlockSpec((1, 1), lambda i: (0, 0)),
        ],
        out_specs=[
            pl.BlockSpec((512, 1), lambda i: (i, 0)),
            pl.BlockSpec((512, 1), lambda i: (i, 0)),
        ],
        out_shape=[
            jax.ShapeDtypeStruct((NP, 1), jnp.float32),
            jax.ShapeDtypeStruct((NP, 1), jnp.float32),
        ],
    )(agg, nd, b, we_top, we_bot, be)


def _norm_body(h_ref, o_ref):
    deg = h_ref[0, 0][:, 0:1]
    o_ref[0, 0] = jnp.where(deg > 0.0,
                            lax.rsqrt(jnp.maximum(deg, 1.0)),
                            0.0)


def _tc_norms(hist):
    return pl.pallas_call(
        _norm_body,
        grid=(3, 2, NP // 512),
        in_specs=[pl.BlockSpec((1, 1, 512, 16), lambda l, d, i: (l, d, i, 0))],
        out_specs=pl.BlockSpec((1, 1, 512, 1), lambda l, d, i: (l, d, i, 0)),
        out_shape=jax.ShapeDtypeStruct((3, 2, NP, 1), jnp.float32),
    )(hist)


def _hist_body(ei_hbm, out_hbm, idx_v, ones_v, zrow_v, acc_sh):
    c = lax.axis_index("c")
    s = lax.axis_index("s")

    one_hot = jnp.where(lax.iota(jnp.int32, 16) == 0, 1.0, 0.0)
    zeros16 = jnp.zeros((16,), jnp.float32)

    @pl.loop(0, HCHUNK)
    def _(i):
        ones_v[i, :] = one_hot

    @pl.loop(0, 224)
    def _(i):
        zrow_v[i, :] = zeros16

    for layer in range(3):
        for j in range(HROWS_PER_SUB // 224):
            pltpu.sync_copy(
                zrow_v, acc_sh.at[pl.ds(s * HROWS_PER_SUB + j * 224, 224)])
        plsc.subcore_barrier()

        base = (layer * 2 + c) * E + s * (E // NSUB)

        @pl.loop(0, E // NSUB, step=HCHUNK)
        def _(off):
            pltpu.sync_copy(ei_hbm.at[pl.ds(base + off, HCHUNK)], idx_v)
            pltpu.sync_copy(ones_v, acc_sh.at[idx_v], add=True)

        plsc.subcore_barrier()

        pltpu.sync_copy(
            acc_sh.at[pl.ds(s * HROWS_PER_SUB, HROWS_PER_SUB)],
            out_hbm.at[layer, c, pl.ds(s * HROWS_PER_SUB, HROWS_PER_SUB)])
        plsc.subcore_barrier()


def _sc_hist(ei_flat):
    kern = pl.kernel(
        _hist_body,
        out_type=jax.ShapeDtypeStruct((3, 2, NP, 16), jnp.float32),
        mesh=_vmesh,
        compiler_params=_sc_params,
        scratch_types=[
            pltpu.VMEM((HCHUNK,), jnp.int32),
            pltpu.VMEM((HCHUNK, 16), jnp.float32),
            pltpu.VMEM((224, 16), jnp.float32),
            pltpu.VMEM_SHARED((NP, 16), jnp.float32),
        ],
    )
    return kern(ei_flat)


def _agg_body(hw_hbm, src_hbm, dst_hbm, out_hbm,
              s_in, d_in, comp_src, comp_loc, srcq, locq, rows_v, zbuf,
              acc_sh, gsem, ssem):
    c = lax.axis_index("c")
    s = lax.axis_index("s")

    RING = 4096
    RMASK = RING - 1
    NCH = (E // NSUB) // ECHUNK

    zeros16f = jnp.zeros((16,), jnp.float32)
    zeros16i = jnp.zeros((16,), jnp.int32)
    dump16 = jnp.full((16,), WN, jnp.int32)

    @pl.loop(0, 8)
    def _(i):
        @pl.loop(0, D, step=16)
        def _(j):
            zbuf[i, pl.ds(j, 16)] = zeros16f

    ebase = s * (E // NSUB)

    def gather_start(q):
        p = q % 3
        o = (q * GQ) & RMASK
        for j in range(GQ // 16):
            srcq[p, pl.ds(j * 16, 16)] = comp_src[pl.ds(o + j * 16, 16)]
            locq[p, pl.ds(j * 16, 16)] = comp_loc[pl.ds(o + j * 16, 16)]
        pltpu.async_copy(hw_hbm.at[srcq.at[p]], rows_v.at[p], gsem.at[p])

    def scat_start(q):
        p = q % 3
        pltpu.make_async_copy(hw_hbm.at[srcq.at[p]], rows_v.at[p],
                              gsem.at[p]).wait()
        pltpu.async_copy(rows_v.at[p], acc_sh.at[locq.at[p]], ssem.at[p],
                         add=True)

    def scat_wait(q):
        p = q % 3
        pltpu.make_async_copy(rows_v.at[p], acc_sh.at[locq.at[p]],
                              ssem.at[p]).wait()

    def fire(q):
        pl.when(q >= 3)(lambda: scat_wait(q - 3))
        gather_start(q)
        pl.when(q >= 2)(lambda: scat_start(q - 2))

    def spill(o):
        @pl.when(o > RING - 16)
        def _():
            comp_src[pl.ds(0, 16)] = comp_src[pl.ds(RING, 16)]
            comp_loc[pl.ds(0, 16)] = comp_loc[pl.ds(RING, 16)]

    @pl.loop(0, WPC)
    def _round(r):
        lo = (c * WPC + r) * WN

        @pl.loop(0, 32)
        def _(j):
            pltpu.sync_copy(zbuf, acc_sh.at[pl.ds(s * 256 + j * 8, 8)])
        plsc.subcore_barrier()

        def chunk(ch, carry):
            n, qg = carry
            pltpu.sync_copy(
                src_hbm.at[pl.ds(ebase + ch * ECHUNK, ECHUNK)], s_in)
            pltpu.sync_copy(
                dst_hbm.at[pl.ds(ebase + ch * ECHUNK, ECHUNK)], d_in)

            def compact(k, n):
                dd = d_in[pl.ds(k * 16, 16)]
                ss = s_in[pl.ds(k * 16, 16)]
                loc = dd - lo
                m = (loc >= 0) & (loc < WN)
                o = n & RMASK
                plsc.store_compressed(comp_src.at[pl.ds(o, 16)], ss, mask=m)
                plsc.store_compressed(comp_loc.at[pl.ds(o, 16)], loc, mask=m)
                spill(o)
                return n + jnp.sum(m.astype(jnp.int32))

            n = lax.fori_loop(0, ECHUNK // 16, compact, n)
            avail = n // GQ - qg

            def floop(i, _):
                fire(qg + i)
                return 0

            lax.fori_loop(0, avail, floop, 0)
            return (n, qg + avail)

        n, qg = lax.fori_loop(0, NCH, chunk, (0, 0))

        def pad_and_fire():
            for j in range(GQ // 16):
                o = (n + j * 16) & RMASK
                comp_src[pl.ds(o, 16)] = zeros16i
                comp_loc[pl.ds(o, 16)] = dump16
                spill(o)
            fire(qg)

        has_tail = (n & (GQ - 1)) > 0
        pl.when(has_tail)(pad_and_fire)
        qt = qg + has_tail.astype(jnp.int32)

        pl.when(qt >= 2)(lambda: scat_start(qt - 2))
        pl.when(qt >= 1)(lambda: scat_start(qt - 1))
        pl.when(qt >= 3)(lambda: scat_wait(qt - 3))
        pl.when(qt >= 2)(lambda: scat_wait(qt - 2))
        pl.when(qt >= 1)(lambda: scat_wait(qt - 1))

        plsc.subcore_barrier()

        pltpu.sync_copy(acc_sh.at[pl.ds(s * 256, 256)],
                        out_hbm.at[pl.ds(lo + s * 256, 256)])
        plsc.subcore_barrier()


def _sc_aggregate(hw, src, dst):
    kern = pl.kernel(
        _agg_body,
        out_type=jax.ShapeDtypeStruct((NP, D), jnp.float32),
        mesh=_vmesh,
        compiler_params=_sc_params,
        scratch_types=[
            pltpu.VMEM((ECHUNK,), jnp.int32),
            pltpu.VMEM((ECHUNK,), jnp.int32),
            pltpu.VMEM((4096 + 16,), jnp.int32),
            pltpu.VMEM((4096 + 16,), jnp.int32),
            pltpu.VMEM((3, GQ), jnp.int32),
            pltpu.VMEM((3, GQ), jnp.int32),
            pltpu.VMEM((3, GQ, D), jnp.float32),
            pltpu.VMEM((8, D), jnp.float32),
            pltpu.VMEM_SHARED((ACC_ROWS, D), jnp.float32),
            pltpu.SemaphoreType.DMA((3,)),
            pltpu.SemaphoreType.DMA((3,)),
        ],
    )
    return kern(hw, src, dst)


def _pred_body(u_hbm, v_hbm, ps_hbm, pd_hbm, out_hbm,
               u_v, v_v, ps_v, pd_v, o_v):
    c = lax.axis_index("c")
    s = lax.axis_index("s")
    w = s * NSC + c

    pltpu.sync_copy(u_hbm, u_v)
    pltpu.sync_copy(v_hbm, v_v)

    base = w * (E // (NSC * NSUB))

    @pl.loop(0, E // (NSC * NSUB), step=HCHUNK)
    def _(off):
        pltpu.sync_copy(ps_hbm.at[pl.ds(base + off, HCHUNK)], ps_v)
        pltpu.sync_copy(pd_hbm.at[pl.ds(base + off, HCHUNK)], pd_v)

        @pl.loop(0, HCHUNK, step=16)
        def _(k):
            pi = ps_v[pl.ds(k, 16)]
            di = pd_v[pl.ds(k, 16)]
            us = plsc.load_gather(u_v, [pi >> 7, pi & 127])
            vs = plsc.load_gather(v_v, [di >> 7, di & 127])
            o_v[pl.ds(k, 16)] = us + vs

        pltpu.sync_copy(o_v, out_hbm.at[pl.ds(base + off, HCHUNK)])


def _sc_pred(u, v, ps, pd):
    kern = pl.kernel(
        _pred_body,
        out_type=jax.ShapeDtypeStruct((E,), jnp.float32),
        mesh=_vmesh,
        compiler_params=_sc_params,
        scratch_types=[
            pltpu.VMEM((NP // 128, 128), jnp.float32),
            pltpu.VMEM((NP // 128, 128), jnp.float32),
            pltpu.VMEM((HCHUNK,), jnp.int32),
            pltpu.VMEM((HCHUNK,), jnp.int32),
            pltpu.VMEM((HCHUNK,), jnp.float32),
        ],
    )
    return kern(u, v, ps, pd)


def kernel(x, edge_index0, edge_index1, edge_index2, pred_edge_index,
           Wr, br, W1, b1, W2, b2, W3, b3, We, be):
    ei_all = jnp.stack([edge_index0.astype(jnp.int32),
                        edge_index1.astype(jnp.int32),
                        edge_index2.astype(jnp.int32)])
    pei = pred_edge_index.astype(jnp.int32)

    hist = _sc_hist(ei_all.reshape(6 * E))
    norms = _tc_norms(hist)

    h0 = _tc_reducer(x, Wr, br.reshape(1, D))

    hw1 = _tc_scale_matmul(h0, norms[0, 0], W1)
    agg1 = _sc_aggregate(hw1, ei_all[0, 0], ei_all[0, 1])

    hw2 = _tc_post_matmul(agg1, norms[0, 1], b1.reshape(1, D), norms[1, 0], W2)
    agg2 = _sc_aggregate(hw2, ei_all[1, 0], ei_all[1, 1])

    hw3 = _tc_post_matmul(agg2, norms[1, 1], b2.reshape(1, D), norms[2, 0], W3)
    agg3 = _sc_aggregate(hw3, ei_all[2, 0], ei_all[2, 1])

    u, v = _tc_uv(agg3, norms[2, 1], b3.reshape(1, D),
                  We[:D, 0].reshape(1, D), We[D:, 0].reshape(1, D),
                  be.reshape(1, 1))

    score = _sc_pred(u.reshape(NP // 128, 128), v.reshape(NP // 128, 128),
                     pei[0], pei[1])
    return score.reshape(E, 1)


# speedup vs baseline: 5.6573x; 1.0006x over previous
import functools

import jax
import jax.numpy as jnp
from jax import lax
from jax.experimental import pallas as pl
from jax.experimental.pallas import tpu as pltpu
from jax.experimental.pallas import tpu_sc as plsc

N = 50000
NP = 57344
E = 1600000
D_IN = 512
D = 256

NSC = 2
NSUB = 16
LANES = 16

WN = 4096
WPC = 7
ACC_ROWS = WN + 8
ECHUNK = 2000
GQ = 64

HCHUNK = 2000
HROWS_PER_SUB = NP // NSUB

_vmesh = plsc.VectorSubcoreMesh(core_axis_name="c", subcore_axis_name="s")
_sc_params = pltpu.CompilerParams(use_tc_tiling_on_sc=False,
                                  needs_layout_passes=False)


def _reducer_body(x_ref, w_ref, b_ref, o_ref):
    o_ref[...] = (
        jnp.dot(x_ref[...], w_ref[...], preferred_element_type=jnp.float32)
        + b_ref[...]
    )


def _tc_reducer(x, wr, br):
    return pl.pallas_call(
        _reducer_body,
        grid=(125,),
        in_specs=[
            pl.BlockSpec((400, D_IN), lambda i: (i, 0)),
            pl.BlockSpec((D_IN, D), lambda i: (0, 0)),
            pl.BlockSpec((1, D), lambda i: (0, 0)),
        ],
        out_specs=pl.BlockSpec((400, D), lambda i: (i, 0)),
        out_shape=jax.ShapeDtypeStruct((NP, D), jnp.float32),
    )(x, wr, br)


def _scale_mm_body(h_ref, ns_ref, w_ref, o_ref):
    o_ref[...] = jnp.dot(
        h_ref[...] * ns_ref[...], w_ref[...],
        preferred_element_type=jnp.float32)


def _tc_scale_matmul(h, ns, w):
    return pl.pallas_call(
        _scale_mm_body,
        grid=(NP // 512,),
        in_specs=[
            pl.BlockSpec((512, D), lambda i: (i, 0)),
            pl.BlockSpec((512, 1), lambda i: (i, 0)),
            pl.BlockSpec((D, D), lambda i: (0, 0)),
        ],
        out_specs=pl.BlockSpec((512, D), lambda i: (i, 0)),
        out_shape=jax.ShapeDtypeStruct((NP, D), jnp.float32),
    )(h, ns, w)


def _post_mm_body(a_ref, nd_ref, b_ref, ns_ref, w_ref, o_ref):
    z = jax.nn.relu(a_ref[...] * nd_ref[...] + b_ref[...])
    o_ref[...] = jnp.dot(z * ns_ref[...], w_ref[...],
                         preferred_element_type=jnp.float32)


def _tc_post_matmul(agg, nd, b, ns, w):
    return pl.pallas_call(
        _post_mm_body,
        grid=(NP // 512,),
        in_specs=[
            pl.BlockSpec((512, D), lambda i: (i, 0)),
            pl.BlockSpec((512, 1), lambda i: (i, 0)),
            pl.BlockSpec((1, D), lambda i: (0, 0)),
            pl.BlockSpec((512, 1), lambda i: (i, 0)),
            pl.BlockSpec((D, D), lambda i: (0, 0)),
        ],
        out_specs=pl.BlockSpec((512, D), lambda i: (i, 0)),
        out_shape=jax.ShapeDtypeStruct((NP, D), jnp.float32),
    )(agg, nd, b, ns, w)


def _uv_body(a_ref, nd_ref, b_ref, wt_ref, wb_ref, be_ref, u_ref, v_ref):
    z = jax.nn.relu(a_ref[...] * nd_ref[...] + b_ref[...])
    u_ref[...] = jnp.sum(z * wt_ref[...], axis=1, keepdims=True) + be_ref[0, 0]
    v_ref[...] = jnp.sum(z * wb_ref[...], axis=1, keepdims=True)


def _tc_uv(agg, nd, b, we_top, we_bot, be):
    return pl.pallas_call(
        _uv_body,
        grid=(NP // 512,),
        in_specs=[
            pl.BlockSpec((512, D), lambda i: (i, 0)),
            pl.BlockSpec((512, 1), lambda i: (i, 0)),
            pl.BlockSpec((1, D), lambda i: (0, 0)),
            pl.BlockSpec((1, D), lambda i: (0, 0)),
            pl.BlockSpec((1, D), lambda i: (0, 0)),
            pl.BlockSpec((1, 1), lambda i: (0, 0)),
        ],
        out_specs=[
            pl.BlockSpec((512, 1), lambda i: (i, 0)),
            pl.BlockSpec((512, 1), lambda i: (i, 0)),
        ],
        out_shape=[
            jax.ShapeDtypeStruct((NP, 1), jnp.float32),
            jax.ShapeDtypeStruct((NP, 1), jnp.float32),
        ],
    )(agg, nd, b, we_top, we_bot, be)


def _norm_body(h_ref, o_ref):
    deg = h_ref[0, 0][:, 0:1]
    o_ref[0, 0] = jnp.where(deg > 0.0,
                            lax.rsqrt(jnp.maximum(deg, 1.0)),
                            0.0)


def _tc_norms(hist):
    return pl.pallas_call(
        _norm_body,
        grid=(3, 2, NP // 512),
        in_specs=[pl.BlockSpec((1, 1, 512, 16), lambda l, d, i: (l, d, i, 0))],
        out_specs=pl.BlockSpec((1, 1, 512, 1), lambda l, d, i: (l, d, i, 0)),
        out_shape=jax.ShapeDtypeStruct((3, 2, NP, 1), jnp.float32),
    )(hist)


def _hist_body(ei_hbm, out_hbm, idx_v, ones_v, zrow_v, acc_sh):
    c = lax.axis_index("c")
    s = lax.axis_index("s")

    one_hot = jnp.where(lax.iota(jnp.int32, 16) == 0, 1.0, 0.0)
    zeros16 = jnp.zeros((16,), jnp.float32)

    @pl.loop(0, HCHUNK)
    def _(i):
        ones_v[i, :] = one_hot

    @pl.loop(0, 224)
    def _(i):
        zrow_v[i, :] = zeros16

    for layer in range(3):
        for j in range(HROWS_PER_SUB // 224):
            pltpu.sync_copy(
                zrow_v, acc_sh.at[pl.ds(s * HROWS_PER_SUB + j * 224, 224)])
        plsc.subcore_barrier()

        base = (layer * 2 + c) * E + s * (E // NSUB)

        @pl.loop(0, E // NSUB, step=HCHUNK)
        def _(off):
            pltpu.sync_copy(ei_hbm.at[pl.ds(base + off, HCHUNK)], idx_v)
            pltpu.sync_copy(ones_v, acc_sh.at[idx_v], add=True)

        plsc.subcore_barrier()

        pltpu.sync_copy(
            acc_sh.at[pl.ds(s * HROWS_PER_SUB, HROWS_PER_SUB)],
            out_hbm.at[layer, c, pl.ds(s * HROWS_PER_SUB, HROWS_PER_SUB)])
        plsc.subcore_barrier()


def _sc_hist(ei_flat):
    kern = pl.kernel(
        _hist_body,
        out_type=jax.ShapeDtypeStruct((3, 2, NP, 16), jnp.float32),
        mesh=_vmesh,
        compiler_params=_sc_params,
        scratch_types=[
            pltpu.VMEM((HCHUNK,), jnp.int32),
            pltpu.VMEM((HCHUNK, 16), jnp.float32),
            pltpu.VMEM((224, 16), jnp.float32),
            pltpu.VMEM_SHARED((NP, 16), jnp.float32),
        ],
    )
    return kern(ei_flat)


def _agg_body(hw_hbm, src_hbm, dst_hbm, out_hbm,
              s_in, d_in, comp_src, comp_loc, srcq, locq, rows_v, zbuf,
              acc_sh, gsem, ssem):
    c = lax.axis_index("c")
    s = lax.axis_index("s")

    RING = 4096
    RMASK = RING - 1
    NCH = (E // NSUB) // ECHUNK

    zeros16f = jnp.zeros((16,), jnp.float32)
    zeros16i = jnp.zeros((16,), jnp.int32)
    dump16 = jnp.full((16,), WN, jnp.int32)

    @pl.loop(0, 8)
    def _(i):
        @pl.loop(0, D, step=16)
        def _(j):
            zbuf[i, pl.ds(j, 16)] = zeros16f

    ebase = s * (E // NSUB)

    def gather_start(q):
        p = q % 3
        o = (q * GQ) & RMASK
        for j in range(GQ // 16):
            srcq[p, pl.ds(j * 16, 16)] = comp_src[pl.ds(o + j * 16, 16)]
            locq[p, pl.ds(j * 16, 16)] = comp_loc[pl.ds(o + j * 16, 16)]
        pltpu.async_copy(hw_hbm.at[srcq.at[p]], rows_v.at[p], gsem.at[p])

    def scat_start(q):
        p = q % 3
        pltpu.make_async_copy(hw_hbm.at[srcq.at[p]], rows_v.at[p],
                              gsem.at[p]).wait()
        pltpu.async_copy(rows_v.at[p], acc_sh.at[locq.at[p]], ssem.at[p],
                         add=True)

    def scat_wait(q):
        p = q % 3
        pltpu.make_async_copy(rows_v.at[p], acc_sh.at[locq.at[p]],
                              ssem.at[p]).wait()

    def fire(q):
        pl.when(q >= 3)(lambda: scat_wait(q - 3))
        gather_start(q)
        pl.when(q >= 2)(lambda: scat_start(q - 2))

    def spill(o):
        @pl.when(o > RING - 16)
        def _():
            comp_src[pl.ds(0, 16)] = comp_src[pl.ds(RING, 16)]
            comp_loc[pl.ds(0, 16)] = comp_loc[pl.ds(RING, 16)]

    @pl.loop(0, WPC)
    def _round(r):
        lo = (c * WPC + r) * WN

        @pl.loop(0, 32)
        def _(j):
            pltpu.sync_copy(zbuf, acc_sh.at[pl.ds(s * 256 + j * 8, 8)])
        plsc.subcore_barrier()

        def chunk(ch, carry):
            n, qg = carry
            pltpu.sync_copy(
                src_hbm.at[pl.ds(ebase + ch * ECHUNK, ECHUNK)], s_in)
            pltpu.sync_copy(
                dst_hbm.at[pl.ds(ebase + ch * ECHUNK, ECHUNK)], d_in)

            def compact(k, n):
                dd = d_in[pl.ds(k * 16, 16)]
                ss = s_in[pl.ds(k * 16, 16)]
                loc = dd - lo
                m = plsc.bitcast(loc, jnp.uint32) < jnp.uint32(WN)
                o = n & RMASK
                plsc.store_compressed(comp_src.at[pl.ds(o, 16)], ss, mask=m)
                plsc.store_compressed(comp_loc.at[pl.ds(o, 16)], loc, mask=m)
                spill(o)
                return n + plsc.all_reduce_population_count(m)[0]

            n = lax.fori_loop(0, ECHUNK // 16, compact, n)
            avail = n // GQ - qg

            def floop(i, _):
                fire(qg + i)
                return 0

            lax.fori_loop(0, avail, floop, 0)
            return (n, qg + avail)

        n, qg = lax.fori_loop(0, NCH, chunk, (0, 0))

        def pad_and_fire():
            for j in range(GQ // 16):
                o = (n + j * 16) & RMASK
                comp_src[pl.ds(o, 16)] = zeros16i
                comp_loc[pl.ds(o, 16)] = dump16
                spill(o)
            fire(qg)

        has_tail = (n & (GQ - 1)) > 0
        pl.when(has_tail)(pad_and_fire)
        qt = qg + has_tail.astype(jnp.int32)

        pl.when(qt >= 2)(lambda: scat_start(qt - 2))
        pl.when(qt >= 1)(lambda: scat_start(qt - 1))
        pl.when(qt >= 3)(lambda: scat_wait(qt - 3))
        pl.when(qt >= 2)(lambda: scat_wait(qt - 2))
        pl.when(qt >= 1)(lambda: scat_wait(qt - 1))

        plsc.subcore_barrier()

        pltpu.sync_copy(acc_sh.at[pl.ds(s * 256, 256)],
                        out_hbm.at[pl.ds(lo + s * 256, 256)])
        plsc.subcore_barrier()


def _sc_aggregate(hw, src, dst):
    kern = pl.kernel(
        _agg_body,
        out_type=jax.ShapeDtypeStruct((NP, D), jnp.float32),
        mesh=_vmesh,
        compiler_params=_sc_params,
        scratch_types=[
            pltpu.VMEM((ECHUNK,), jnp.int32),
            pltpu.VMEM((ECHUNK,), jnp.int32),
            pltpu.VMEM((4096 + 16,), jnp.int32),
            pltpu.VMEM((4096 + 16,), jnp.int32),
            pltpu.VMEM((3, GQ), jnp.int32),
            pltpu.VMEM((3, GQ), jnp.int32),
            pltpu.VMEM((3, GQ, D), jnp.float32),
            pltpu.VMEM((8, D), jnp.float32),
            pltpu.VMEM_SHARED((ACC_ROWS, D), jnp.float32),
            pltpu.SemaphoreType.DMA((3,)),
            pltpu.SemaphoreType.DMA((3,)),
        ],
    )
    return kern(hw, src, dst)


def _pred_body(u_hbm, v_hbm, ps_hbm, pd_hbm, out_hbm,
               u_v, v_v, ps_v, pd_v, o_v):
    c = lax.axis_index("c")
    s = lax.axis_index("s")
    w = s * NSC + c

    pltpu.sync_copy(u_hbm, u_v)
    pltpu.sync_copy(v_hbm, v_v)

    base = w * (E // (NSC * NSUB))

    @pl.loop(0, E // (NSC * NSUB), step=HCHUNK)
    def _(off):
        pltpu.sync_copy(ps_hbm.at[pl.ds(base + off, HCHUNK)], ps_v)
        pltpu.sync_copy(pd_hbm.at[pl.ds(base + off, HCHUNK)], pd_v)

        @pl.loop(0, HCHUNK, step=16)
        def _(k):
            pi = ps_v[pl.ds(k, 16)]
            di = pd_v[pl.ds(k, 16)]
            us = plsc.load_gather(u_v, [pi >> 7, pi & 127])
            vs = plsc.load_gather(v_v, [di >> 7, di & 127])
            o_v[pl.ds(k, 16)] = us + vs

        pltpu.sync_copy(o_v, out_hbm.at[pl.ds(base + off, HCHUNK)])


def _sc_pred(u, v, ps, pd):
    kern = pl.kernel(
        _pred_body,
        out_type=jax.ShapeDtypeStruct((E,), jnp.float32),
        mesh=_vmesh,
        compiler_params=_sc_params,
        scratch_types=[
            pltpu.VMEM((NP // 128, 128), jnp.float32),
            pltpu.VMEM((NP // 128, 128), jnp.float32),
            pltpu.VMEM((HCHUNK,), jnp.int32),
            pltpu.VMEM((HCHUNK,), jnp.int32),
            pltpu.VMEM((HCHUNK,), jnp.float32),
        ],
    )
    return kern(u, v, ps, pd)


def kernel(x, edge_index0, edge_index1, edge_index2, pred_edge_index,
           Wr, br, W1, b1, W2, b2, W3, b3, We, be):
    ei_all = jnp.stack([edge_index0.astype(jnp.int32),
                        edge_index1.astype(jnp.int32),
                        edge_index2.astype(jnp.int32)])
    pei = pred_edge_index.astype(jnp.int32)

    hist = _sc_hist(ei_all.reshape(6 * E))
    norms = _tc_norms(hist)

    h0 = _tc_reducer(x, Wr, br.reshape(1, D))

    hw1 = _tc_scale_matmul(h0, norms[0, 0], W1)
    agg1 = _sc_aggregate(hw1, ei_all[0, 0], ei_all[0, 1])

    hw2 = _tc_post_matmul(agg1, norms[0, 1], b1.reshape(1, D), norms[1, 0], W2)
    agg2 = _sc_aggregate(hw2, ei_all[1, 0], ei_all[1, 1])

    hw3 = _tc_post_matmul(agg2, norms[1, 1], b2.reshape(1, D), norms[2, 0], W3)
    agg3 = _sc_aggregate(hw3, ei_all[2, 0], ei_all[2, 1])

    u, v = _tc_uv(agg3, norms[2, 1], b3.reshape(1, D),
                  We[:D, 0].reshape(1, D), We[D:, 0].reshape(1, D),
                  be.reshape(1, 1))

    score = _sc_pred(u.reshape(NP // 128, 128), v.reshape(NP // 128, 128),
                     pei[0], pei[1])
    return score.reshape(E, 1)


# speedup vs baseline: 7.6636x; 1.3546x over previous
import functools

import jax
import jax.numpy as jnp
from jax import lax
from jax.experimental import pallas as pl
from jax.experimental.pallas import tpu as pltpu
from jax.experimental.pallas import tpu_sc as plsc

N = 50000
NP = 57344
E = 1600000
D_IN = 512
D = 256

NSC = 2
NSUB = 16
LANES = 16

WN = 4096
WPC = 7
ACC_ROWS = WN + 8
ECHUNK = 800
GQ = 64

HCHUNK = 2000
HROWS_PER_SUB = NP // NSUB

_vmesh = plsc.VectorSubcoreMesh(core_axis_name="c", subcore_axis_name="s")
_sc_params = pltpu.CompilerParams(use_tc_tiling_on_sc=False,
                                  needs_layout_passes=False)


def _reducer_body(x_ref, w_ref, b_ref, o_ref):
    o_ref[...] = (
        jnp.dot(x_ref[...], w_ref[...], preferred_element_type=jnp.float32)
        + b_ref[...]
    )


def _tc_reducer(x, wr, br):
    return pl.pallas_call(
        _reducer_body,
        grid=(125,),
        in_specs=[
            pl.BlockSpec((400, D_IN), lambda i: (i, 0)),
            pl.BlockSpec((D_IN, D), lambda i: (0, 0)),
            pl.BlockSpec((1, D), lambda i: (0, 0)),
        ],
        out_specs=pl.BlockSpec((400, D), lambda i: (i, 0)),
        out_shape=jax.ShapeDtypeStruct((NP, D), jnp.float32),
    )(x, wr, br)


def _scale_mm_body(h_ref, ns_ref, w_ref, o_ref):
    o_ref[...] = jnp.dot(
        h_ref[...] * ns_ref[...], w_ref[...],
        preferred_element_type=jnp.float32)


def _tc_scale_matmul(h, ns, w):
    return pl.pallas_call(
        _scale_mm_body,
        grid=(NP // 512,),
        in_specs=[
            pl.BlockSpec((512, D), lambda i: (i, 0)),
            pl.BlockSpec((512, 1), lambda i: (i, 0)),
            pl.BlockSpec((D, D), lambda i: (0, 0)),
        ],
        out_specs=pl.BlockSpec((512, D), lambda i: (i, 0)),
        out_shape=jax.ShapeDtypeStruct((NP, D), jnp.float32),
    )(h, ns, w)


def _post_mm_body(a_ref, nd_ref, b_ref, ns_ref, w_ref, o_ref):
    z = jax.nn.relu(a_ref[...] * nd_ref[...] + b_ref[...])
    o_ref[...] = jnp.dot(z * ns_ref[...], w_ref[...],
                         preferred_element_type=jnp.float32)


def _tc_post_matmul(agg, nd, b, ns, w):
    return pl.pallas_call(
        _post_mm_body,
        grid=(NP // 512,),
        in_specs=[
            pl.BlockSpec((512, D), lambda i: (i, 0)),
            pl.BlockSpec((512, 1), lambda i: (i, 0)),
            pl.BlockSpec((1, D), lambda i: (0, 0)),
            pl.BlockSpec((512, 1), lambda i: (i, 0)),
            pl.BlockSpec((D, D), lambda i: (0, 0)),
        ],
        out_specs=pl.BlockSpec((512, D), lambda i: (i, 0)),
        out_shape=jax.ShapeDtypeStruct((NP, D), jnp.float32),
    )(agg, nd, b, ns, w)


def _uv_body(a_ref, nd_ref, b_ref, wt_ref, wb_ref, be_ref, u_ref, v_ref):
    z = jax.nn.relu(a_ref[...] * nd_ref[...] + b_ref[...])
    u_ref[...] = jnp.sum(z * wt_ref[...], axis=1, keepdims=True) + be_ref[0, 0]
    v_ref[...] = jnp.sum(z * wb_ref[...], axis=1, keepdims=True)


def _tc_uv(agg, nd, b, we_top, we_bot, be):
    return pl.pallas_call(
        _uv_body,
        grid=(NP // 512,),
        in_specs=[
            pl.BlockSpec((512, D), lambda i: (i, 0)),
            pl.BlockSpec((512, 1), lambda i: (i, 0)),
            pl.BlockSpec((1, D), lambda i: (0, 0)),
            pl.BlockSpec((1, D), lambda i: (0, 0)),
            pl.BlockSpec((1, D), lambda i: (0, 0)),
            pl.BlockSpec((1, 1), lambda i: (0, 0)),
        ],
        out_specs=[
            pl.BlockSpec((512, 1), lambda i: (i, 0)),
            pl.BlockSpec((512, 1), lambda i: (i, 0)),
        ],
        out_shape=[
            jax.ShapeDtypeStruct((NP, 1), jnp.float32),
            jax.ShapeDtypeStruct((NP, 1), jnp.float32),
        ],
    )(agg, nd, b, we_top, we_bot, be)


def _norm_body(h_ref, o_ref):
    deg = h_ref[0, 0][:, 0:1]
    o_ref[0, 0] = jnp.where(deg > 0.0,
                            lax.rsqrt(jnp.maximum(deg, 1.0)),
                            0.0)


def _tc_norms(hist):
    return pl.pallas_call(
        _norm_body,
        grid=(3, 2, NP // 512),
        in_specs=[pl.BlockSpec((1, 1, 512, 16), lambda l, d, i: (l, d, i, 0))],
        out_specs=pl.BlockSpec((1, 1, 512, 1), lambda l, d, i: (l, d, i, 0)),
        out_shape=jax.ShapeDtypeStruct((3, 2, NP, 1), jnp.float32),
    )(hist)


def _hist_body(ei_hbm, out_hbm, idx_v, ones_v, zrow_v, acc_sh):
    c = lax.axis_index("c")
    s = lax.axis_index("s")

    one_hot = jnp.where(lax.iota(jnp.int32, 16) == 0, 1.0, 0.0)
    zeros16 = jnp.zeros((16,), jnp.float32)

    @pl.loop(0, HCHUNK)
    def _(i):
        ones_v[i, :] = one_hot

    @pl.loop(0, 224)
    def _(i):
        zrow_v[i, :] = zeros16

    for layer in range(3):
        for j in range(HROWS_PER_SUB // 224):
            pltpu.sync_copy(
                zrow_v, acc_sh.at[pl.ds(s * HROWS_PER_SUB + j * 224, 224)])
        plsc.subcore_barrier()

        base = (layer * 2 + c) * E + s * (E // NSUB)

        @pl.loop(0, E // NSUB, step=HCHUNK)
        def _(off):
            pltpu.sync_copy(ei_hbm.at[pl.ds(base + off, HCHUNK)], idx_v)
            pltpu.sync_copy(ones_v, acc_sh.at[idx_v], add=True)

        plsc.subcore_barrier()

        pltpu.sync_copy(
            acc_sh.at[pl.ds(s * HROWS_PER_SUB, HROWS_PER_SUB)],
            out_hbm.at[layer, c, pl.ds(s * HROWS_PER_SUB, HROWS_PER_SUB)])
        plsc.subcore_barrier()


def _sc_hist(ei_flat):
    kern = pl.kernel(
        _hist_body,
        out_type=jax.ShapeDtypeStruct((3, 2, NP, 16), jnp.float32),
        mesh=_vmesh,
        compiler_params=_sc_params,
        scratch_types=[
            pltpu.VMEM((HCHUNK,), jnp.int32),
            pltpu.VMEM((HCHUNK, 16), jnp.float32),
            pltpu.VMEM((224, 16), jnp.float32),
            pltpu.VMEM_SHARED((NP, 16), jnp.float32),
        ],
    )
    return kern(ei_flat)


def _agg_body(hw_hbm, src_hbm, dst_hbm, out_hbm,
              s_in, d_in, comp_src, comp_loc, srcq, locq, rows_v, zbuf,
              acc_sh, gsem, ssem, isem):
    c = lax.axis_index("c")
    s = lax.axis_index("s")

    RING = 1024
    RMASK = RING - 1
    NCH = (E // NSUB) // ECHUNK

    zeros16f = jnp.zeros((16,), jnp.float32)
    zeros16i = jnp.zeros((16,), jnp.int32)
    dump16 = jnp.full((16,), WN, jnp.int32)

    @pl.loop(0, 8)
    def _(i):
        @pl.loop(0, D, step=16)
        def _(j):
            zbuf[i, pl.ds(j, 16)] = zeros16f

    ebase = s * (E // NSUB)

    def gather_start(q):
        p = q % 3
        o = (q * GQ) & RMASK
        for j in range(GQ // 16):
            srcq[p, pl.ds(j * 16, 16)] = comp_src[pl.ds(o + j * 16, 16)]
            locq[p, pl.ds(j * 16, 16)] = comp_loc[pl.ds(o + j * 16, 16)]
        pltpu.async_copy(hw_hbm.at[srcq.at[p]], rows_v.at[p], gsem.at[p])

    def scat_start(q):
        p = q % 3
        pltpu.make_async_copy(hw_hbm.at[srcq.at[p]], rows_v.at[p],
                              gsem.at[p]).wait()
        pltpu.async_copy(rows_v.at[p], acc_sh.at[locq.at[p]], ssem.at[p],
                         add=True)

    def scat_wait(q):
        p = q % 3
        pltpu.make_async_copy(rows_v.at[p], acc_sh.at[locq.at[p]],
                              ssem.at[p]).wait()

    def fire(q):
        pl.when(q >= 3)(lambda: scat_wait(q - 3))
        gather_start(q)
        pl.when(q >= 2)(lambda: scat_start(q - 2))

    def spill(o):
        @pl.when(o > RING - 16)
        def _():
            comp_src[pl.ds(0, 16)] = comp_src[pl.ds(RING, 16)]
            comp_loc[pl.ds(0, 16)] = comp_loc[pl.ds(RING, 16)]

    @pl.loop(0, WPC)
    def _round(r):
        lo = (c * WPC + r) * WN

        @pl.loop(0, 32)
        def _(j):
            pltpu.sync_copy(zbuf, acc_sh.at[pl.ds(s * 256 + j * 8, 8)])
        plsc.subcore_barrier()

        def in_start(ch):
            pb = ch & 1
            off = ebase + ch * ECHUNK
            pltpu.async_copy(src_hbm.at[pl.ds(off, ECHUNK)], s_in.at[pb],
                             isem.at[pb])
            pltpu.async_copy(dst_hbm.at[pl.ds(off, ECHUNK)], d_in.at[pb],
                             isem.at[pb])

        def in_wait(ch):
            pb = ch & 1
            off = ebase + ch * ECHUNK
            pltpu.make_async_copy(src_hbm.at[pl.ds(off, ECHUNK)],
                                  s_in.at[pb], isem.at[pb]).wait()
            pltpu.make_async_copy(dst_hbm.at[pl.ds(off, ECHUNK)],
                                  d_in.at[pb], isem.at[pb]).wait()

        in_start(0)

        def chunk(ch, carry):
            n, qg = carry
            pb = ch & 1
            in_wait(ch)
            pl.when(ch + 1 < NCH)(lambda: in_start(ch + 1))

            def compact(k, n):
                dd = d_in[pb, pl.ds(k * 16, 16)]
                ss = s_in[pb, pl.ds(k * 16, 16)]
                loc = dd - lo
                m = plsc.bitcast(loc, jnp.uint32) < jnp.uint32(WN)
                o = n & RMASK
                plsc.store_compressed(comp_src.at[pl.ds(o, 16)], ss, mask=m)
                plsc.store_compressed(comp_loc.at[pl.ds(o, 16)], loc, mask=m)
                spill(o)
                return n + plsc.all_reduce_population_count(m)[0]

            n = lax.fori_loop(0, ECHUNK // 16, compact, n)
            avail = n // GQ - qg

            def floop(i, _):
                fire(qg + i)
                return 0

            lax.fori_loop(0, avail, floop, 0)
            return (n, qg + avail)

        n, qg = lax.fori_loop(0, NCH, chunk, (0, 0))

        def pad_and_fire():
            for j in range(GQ // 16):
                o = (n + j * 16) & RMASK
                comp_src[pl.ds(o, 16)] = zeros16i
                comp_loc[pl.ds(o, 16)] = dump16
                spill(o)
            fire(qg)

        has_tail = (n & (GQ - 1)) > 0
        pl.when(has_tail)(pad_and_fire)
        qt = qg + has_tail.astype(jnp.int32)

        pl.when(qt >= 2)(lambda: scat_start(qt - 2))
        pl.when(qt >= 1)(lambda: scat_start(qt - 1))
        pl.when(qt >= 3)(lambda: scat_wait(qt - 3))
        pl.when(qt >= 2)(lambda: scat_wait(qt - 2))
        pl.when(qt >= 1)(lambda: scat_wait(qt - 1))

        plsc.subcore_barrier()

        pltpu.sync_copy(acc_sh.at[pl.ds(s * 256, 256)],
                        out_hbm.at[pl.ds(lo + s * 256, 256)])
        plsc.subcore_barrier()


def _sc_aggregate(hw, src, dst):
    kern = pl.kernel(
        _agg_body,
        out_type=jax.ShapeDtypeStruct((NP, D), jnp.float32),
        mesh=_vmesh,
        compiler_params=_sc_params,
        scratch_types=[
            pltpu.VMEM((2, ECHUNK), jnp.int32),
            pltpu.VMEM((2, ECHUNK), jnp.int32),
            pltpu.VMEM((1024 + 16,), jnp.int32),
            pltpu.VMEM((1024 + 16,), jnp.int32),
            pltpu.VMEM((3, GQ), jnp.int32),
            pltpu.VMEM((3, GQ), jnp.int32),
            pltpu.VMEM((3, GQ, D), jnp.float32),
            pltpu.VMEM((8, D), jnp.float32),
            pltpu.VMEM_SHARED((ACC_ROWS, D), jnp.float32),
            pltpu.SemaphoreType.DMA((3,)),
            pltpu.SemaphoreType.DMA((3,)),
            pltpu.SemaphoreType.DMA((2,)),
        ],
    )
    return kern(hw, src, dst)


def _pred_body(u_hbm, v_hbm, ps_hbm, pd_hbm, out_hbm,
               u_v, v_v, ps_v, pd_v, o_v):
    c = lax.axis_index("c")
    s = lax.axis_index("s")
    w = s * NSC + c

    pltpu.sync_copy(u_hbm, u_v)
    pltpu.sync_copy(v_hbm, v_v)

    base = w * (E // (NSC * NSUB))

    @pl.loop(0, E // (NSC * NSUB), step=HCHUNK)
    def _(off):
        pltpu.sync_copy(ps_hbm.at[pl.ds(base + off, HCHUNK)], ps_v)
        pltpu.sync_copy(pd_hbm.at[pl.ds(base + off, HCHUNK)], pd_v)

        @pl.loop(0, HCHUNK, step=16)
        def _(k):
            pi = ps_v[pl.ds(k, 16)]
            di = pd_v[pl.ds(k, 16)]
            us = plsc.load_gather(u_v, [pi >> 7, pi & 127])
            vs = plsc.load_gather(v_v, [di >> 7, di & 127])
            o_v[pl.ds(k, 16)] = us + vs

        pltpu.sync_copy(o_v, out_hbm.at[pl.ds(base + off, HCHUNK)])


def _sc_pred(u, v, ps, pd):
    kern = pl.kernel(
        _pred_body,
        out_type=jax.ShapeDtypeStruct((E,), jnp.float32),
        mesh=_vmesh,
        compiler_params=_sc_params,
        scratch_types=[
            pltpu.VMEM((NP // 128, 128), jnp.float32),
            pltpu.VMEM((NP // 128, 128), jnp.float32),
            pltpu.VMEM((HCHUNK,), jnp.int32),
            pltpu.VMEM((HCHUNK,), jnp.int32),
            pltpu.VMEM((HCHUNK,), jnp.float32),
        ],
    )
    return kern(u, v, ps, pd)


def kernel(x, edge_index0, edge_index1, edge_index2, pred_edge_index,
           Wr, br, W1, b1, W2, b2, W3, b3, We, be):
    ei_all = jnp.stack([edge_index0.astype(jnp.int32),
                        edge_index1.astype(jnp.int32),
                        edge_index2.astype(jnp.int32)])
    pei = pred_edge_index.astype(jnp.int32)

    hist = _sc_hist(ei_all.reshape(6 * E))
    norms = _tc_norms(hist)

    h0 = _tc_reducer(x, Wr, br.reshape(1, D))

    hw1 = _tc_scale_matmul(h0, norms[0, 0], W1)
    agg1 = _sc_aggregate(hw1, ei_all[0, 0], ei_all[0, 1])

    hw2 = _tc_post_matmul(agg1, norms[0, 1], b1.reshape(1, D), norms[1, 0], W2)
    agg2 = _sc_aggregate(hw2, ei_all[1, 0], ei_all[1, 1])

    hw3 = _tc_post_matmul(agg2, norms[1, 1], b2.reshape(1, D), norms[2, 0], W3)
    agg3 = _sc_aggregate(hw3, ei_all[2, 0], ei_all[2, 1])

    u, v = _tc_uv(agg3, norms[2, 1], b3.reshape(1, D),
                  We[:D, 0].reshape(1, D), We[D:, 0].reshape(1, D),
                  be.reshape(1, 1))

    score = _sc_pred(u.reshape(NP // 128, 128), v.reshape(NP // 128, 128),
                     pei[0], pei[1])
    return score.reshape(E, 1)
